# baseline (speedup 1.0000x reference)
import functools

import jax
import jax.numpy as jnp
from jax import lax
from jax.experimental import pallas as pl
from jax.experimental.pallas import tpu as pltpu
from jax.experimental.pallas import tpu_sc as plsc

_VOCAB = 100000
_D = 16
_NSUB = 12
_E = _D * _NSUB
_BASE = 10000.0
_B, _T = 1024, 200
_BT = _B * _T

_PITCH_SIZE = 128
_POS_SIZE = 32
_BAR_ROWS = _T
_AUX_ROWS = _PITCH_SIZE + _POS_SIZE + _BAR_ROWS

_NC = 2
_NSC = 16
_NW = _NC * _NSC
_ROWS_PER_W = _B // _NW

_CA = 112
_CB = 88
_GROUPS = 13


def _aux_body(bias_ref, out_ref):
    rows = lax.broadcasted_iota(jnp.int32, (_AUX_ROWS, _E), 0)
    cols = lax.broadcasted_iota(jnp.int32, (_AUX_ROWS, _E), 1)
    d = cols % _D
    exponent = 2.0 * jnp.floor(d.astype(jnp.float32) / 2.0) / float(_D)
    rate = jnp.exp(exponent * (-jnp.log(jnp.float32(_BASE))))
    is_pos = (rows >= _PITCH_SIZE) & (rows < _PITCH_SIZE + _POS_SIZE)
    is_bar = rows >= _PITCH_SIZE + _POS_SIZE
    val = jnp.where(
        is_bar, rows - (_PITCH_SIZE + _POS_SIZE),
        jnp.where(is_pos, rows - _PITCH_SIZE, rows)).astype(jnp.float32)
    ang = val * rate
    enc = jnp.where(d % 2 == 0, jnp.sin(ang), jnp.cos(ang))
    pitch_b = bias_ref[0:1, :]
    pos_b = bias_ref[1:2, :]
    bar_b = bias_ref[2:3, :]
    bias = jnp.where(is_bar, bar_b, jnp.where(is_pos, pos_b, pitch_b))
    out_ref[:, :] = enc + bias


def _build_aux(biases):
    return pl.pallas_call(
        _aux_body,
        out_shape=jax.ShapeDtypeStruct((_AUX_ROWS, _E), jnp.float32),
    )(biases)


def _sc_body(idx_hbm, table_hbm, aux_hbm, out_hbm,
             idx_a, idx_b, rows_a, rows_b, auxbuf, sem):
    cid = lax.axis_index("c")
    sid = lax.axis_index("s")
    wid = sid * _NC + cid
    lanes = lax.iota(jnp.int32, 16)

    idx_b[pl.ds(80, 16)] = jnp.where(lanes < 8, idx_b[pl.ds(80, 16)], 0)

    def row_body(r, _):
        row = wid * _ROWS_PER_W + r
        base = pl.multiple_of(row * _T, 8)

        pltpu.sync_copy(idx_hbm.at[pl.ds(base, _CA)], idx_a)
        pltpu.sync_copy(idx_hbm.at[pl.ds(base + _CA, _CB)],
                        idx_b.at[pl.ds(0, _CB)])

        pltpu.async_copy(table_hbm.at[idx_a], rows_a, sem).wait()
        pltpu.async_copy(table_hbm.at[idx_b], rows_b, sem).wait()

        pltpu.sync_copy(rows_a, out_hbm.at[pl.ds(base, _CA)])
        pltpu.sync_copy(rows_b.at[pl.ds(0, _CB)],
                        out_hbm.at[pl.ds(base + _CA, _CB)])

        carry = jnp.int32(0)
        for g in range(_GROUPS):
            if g < 7:
                iv = idx_a[pl.ds(g * 16, 16)]
                nval = 16
            else:
                iv = idx_b[pl.ds((g - 7) * 16, 16)]
                nval = 16 if g < 12 else 8
            valid = lanes < nval
            barm = (iv == 128) & valid
            bar_ones = jnp.where(barm, 1, 0)
            bar_idx = carry + jnp.cumsum(bar_ones) - 1
            carry = carry + jnp.sum(bar_ones)
            maskv = (iv < 161) & valid
            auxi = jnp.where(iv == 128, 160 + bar_idx,
                             jnp.where(iv < 128, iv, iv - 1))
            posg = base + g * 16 + lanes
            cnt = jnp.sum(jnp.where(maskv, 1, 0))

            @pl.when(cnt > 0)
            def _():
                lanestar = jnp.max(jnp.where(maskv, lanes, -1))
                onehot = jnp.where(lanes == lanestar, 1, 0)
                astar = jnp.sum(onehot * auxi)
                pstar = jnp.sum(onehot * posg)
                auxf = jnp.where(maskv, auxi, astar)
                posf = jnp.where(maskv, posg, pstar)
                pltpu.async_copy(aux_hbm.at[auxf], auxbuf, sem).wait()
                pltpu.async_copy(auxbuf, out_hbm.at[posf], sem).wait()

        return 0

    lax.fori_loop(0, _ROWS_PER_W, row_body, 0)


_sc_gather = functools.partial(
    pl.kernel,
    mesh=plsc.VectorSubcoreMesh(core_axis_name="c", subcore_axis_name="s"),
    out_type=jax.ShapeDtypeStruct((_BT, _E), jnp.float32),
    compiler_params=pltpu.CompilerParams(
        needs_layout_passes=False, use_tc_tiling_on_sc=False),
    scratch_types=[
        pltpu.VMEM((_CA,), jnp.int32),
        pltpu.VMEM((_CB + 8,), jnp.int32),
        pltpu.VMEM((_CA, _E), jnp.float32),
        pltpu.VMEM((_CB + 8, _E), jnp.float32),
        pltpu.VMEM((16, _E), jnp.float32),
        pltpu.SemaphoreType.DMA,
    ],
)(_sc_body)


def kernel(idx, table, pitch_bias, pos_bias, bar_bias):
    biases = jnp.concatenate([
        pitch_bias.reshape(1, _E),
        pos_bias.reshape(1, _E),
        bar_bias.reshape(1, _E),
    ], axis=0)
    aux = _build_aux(biases)
    out = _sc_gather(idx.reshape(_BT).astype(jnp.int32), table, aux)
    return out.reshape(_B, _T, _E)


# speedup vs baseline: 1.4294x; 1.4294x over previous
import functools

import jax
import jax.numpy as jnp
from jax import lax
from jax.experimental import pallas as pl
from jax.experimental.pallas import tpu as pltpu
from jax.experimental.pallas import tpu_sc as plsc

_VOCAB = 100000
_D = 16
_NSUB = 12
_E = _D * _NSUB
_BASE = 10000.0
_B, _T = 1024, 200
_BT = _B * _T

_PITCH_SIZE = 128
_POS_SIZE = 32
_BAR_ROWS = _T
_AUX_ROWS = _PITCH_SIZE + _POS_SIZE + _BAR_ROWS

_NC = 2
_NSC = 16
_NW = _NC * _NSC
_ROWS_PER_W = _B // _NW

_CA = 112
_CB = 88
_GROUPS = 13


def _aux_body(bias_ref, out_ref):
    rows = lax.broadcasted_iota(jnp.int32, (_AUX_ROWS, _E), 0)
    cols = lax.broadcasted_iota(jnp.int32, (_AUX_ROWS, _E), 1)
    d = cols % _D
    exponent = 2.0 * jnp.floor(d.astype(jnp.float32) / 2.0) / float(_D)
    rate = jnp.exp(exponent * (-jnp.log(jnp.float32(_BASE))))
    is_pos = (rows >= _PITCH_SIZE) & (rows < _PITCH_SIZE + _POS_SIZE)
    is_bar = rows >= _PITCH_SIZE + _POS_SIZE
    val = jnp.where(
        is_bar, rows - (_PITCH_SIZE + _POS_SIZE),
        jnp.where(is_pos, rows - _PITCH_SIZE, rows)).astype(jnp.float32)
    ang = val * rate
    enc = jnp.where(d % 2 == 0, jnp.sin(ang), jnp.cos(ang))
    pitch_b = bias_ref[0:1, :]
    pos_b = bias_ref[1:2, :]
    bar_b = bias_ref[2:3, :]
    bias = jnp.where(is_bar, bar_b, jnp.where(is_pos, pos_b, pitch_b))
    out_ref[:, :] = enc + bias


def _build_aux(biases):
    return pl.pallas_call(
        _aux_body,
        out_shape=jax.ShapeDtypeStruct((_AUX_ROWS, _E), jnp.float32),
    )(biases)


_TOK_W = _T * _ROWS_PER_W
_CHUNK = 128
_NCHUNK = _TOK_W // _CHUNK
_NBUF = 4


def _sc_body(idx_hbm, table_hbm, aux_hbm, out_hbm,
             idx_all, rb0, rb1, rb2, rb3, auxbuf,
             gs0, gs1, gs2, gs3, ws0, ws1, ws2, ws3, osem):
    bufs = (rb0, rb1, rb2, rb3)
    gsems = (gs0, gs1, gs2, gs3)
    wsems = (ws0, ws1, ws2, ws3)
    cid = lax.axis_index("c")
    sid = lax.axis_index("s")
    wid = sid * _NC + cid
    lanes = lax.iota(jnp.int32, 16)
    tok0 = pl.multiple_of(wid * _TOK_W, 8)

    pltpu.sync_copy(idx_hbm.at[pl.ds(tok0, _TOK_W)],
                    idx_all.at[pl.ds(0, _TOK_W)])
    idx_all[pl.ds(_TOK_W, 16)] = jnp.zeros((16,), jnp.int32)

    def g_start(c, s):
        pltpu.async_copy(
            table_hbm.at[idx_all.at[pl.ds(c * _CHUNK, _CHUNK)]],
            bufs[s], gsems[s])

    def g_wait(s):
        pltpu.make_async_copy(
            table_hbm.at[idx_all.at[pl.ds(0, _CHUNK)]],
            bufs[s], gsems[s]).wait()

    def w_start(c, s):
        pltpu.async_copy(
            bufs[s], out_hbm.at[pl.ds(tok0 + c * _CHUNK, _CHUNK)], wsems[s])

    def w_wait(s):
        pltpu.make_async_copy(
            bufs[s], out_hbm.at[pl.ds(tok0, _CHUNK)], wsems[s]).wait()

    g_start(0, 0)
    g_start(1, 1)

    def chunk4_body(c4, _):
        for s in range(_NBUF):
            d = c4 * _NBUF + s
            s2 = (s + 2) % _NBUF

            @pl.when(d >= 2)
            def _():
                w_wait(s2)

            @pl.when(d + 2 < _NCHUNK)
            def _():
                g_start(d + 2, s2)

            g_wait(s)
            w_start(d, s)
        return 0

    lax.fori_loop(0, _NCHUNK // _NBUF, chunk4_body, 0)
    for d in range(_NCHUNK - _NCHUNK % _NBUF, _NCHUNK):
        s = d % _NBUF
        s2 = (s + 2) % _NBUF
        w_wait(s2)
        g_wait(s)
        w_start(d, s)
    for d in range(_NCHUNK - 2, _NCHUNK):
        w_wait(d % _NBUF)

    def row_body(r, _):
        rb = pl.multiple_of(r * _T, 8)
        carry = jnp.int32(0)
        for g in range(_GROUPS):
            iv = idx_all[pl.ds(rb + g * 16, 16)]
            valid = lanes < (16 if g < 12 else 8)
            barm = (iv == 128) & valid
            bar_ones = jnp.where(barm, 1, 0)
            bar_idx = carry + jnp.cumsum(bar_ones) - 1
            carry = carry + jnp.sum(bar_ones)
            maskv = (iv < 161) & valid
            auxi = jnp.where(iv == 128, 160 + bar_idx,
                             jnp.where(iv < 128, iv, iv - 1))
            posg = tok0 + rb + g * 16 + lanes
            cnt = jnp.sum(jnp.where(maskv, 1, 0))

            @pl.when(cnt > 0)
            def _():
                lanestar = jnp.max(jnp.where(maskv, lanes, -1))
                onehot = jnp.where(lanes == lanestar, 1, 0)
                astar = jnp.sum(onehot * auxi)
                pstar = jnp.sum(onehot * posg)
                auxf = jnp.where(maskv, auxi, astar)
                posf = jnp.where(maskv, posg, pstar)
                pltpu.async_copy(aux_hbm.at[auxf], auxbuf, osem).wait()
                pltpu.async_copy(auxbuf, out_hbm.at[posf], osem).wait()
        return 0

    lax.fori_loop(0, _ROWS_PER_W, row_body, 0)


_sc_gather = functools.partial(
    pl.kernel,
    mesh=plsc.VectorSubcoreMesh(core_axis_name="c", subcore_axis_name="s"),
    out_type=jax.ShapeDtypeStruct((_BT, _E), jnp.float32),
    compiler_params=pltpu.CompilerParams(
        needs_layout_passes=False, use_tc_tiling_on_sc=False),
    scratch_types=[
        pltpu.VMEM((_TOK_W + 16,), jnp.int32),
        pltpu.VMEM((_CHUNK, _E), jnp.float32),
        pltpu.VMEM((_CHUNK, _E), jnp.float32),
        pltpu.VMEM((_CHUNK, _E), jnp.float32),
        pltpu.VMEM((_CHUNK, _E), jnp.float32),
        pltpu.VMEM((16, _E), jnp.float32),
        pltpu.SemaphoreType.DMA,
        pltpu.SemaphoreType.DMA,
        pltpu.SemaphoreType.DMA,
        pltpu.SemaphoreType.DMA,
        pltpu.SemaphoreType.DMA,
        pltpu.SemaphoreType.DMA,
        pltpu.SemaphoreType.DMA,
        pltpu.SemaphoreType.DMA,
        pltpu.SemaphoreType.DMA,
    ],
)(_sc_body)


def kernel(idx, table, pitch_bias, pos_bias, bar_bias):
    biases = jnp.concatenate([
        pitch_bias.reshape(1, _E),
        pos_bias.reshape(1, _E),
        bar_bias.reshape(1, _E),
    ], axis=0)
    aux = _build_aux(biases)
    out = _sc_gather(idx.reshape(_BT).astype(jnp.int32), table, aux)
    return out.reshape(_B, _T, _E)


# speedup vs baseline: 2.4733x; 1.7303x over previous
import functools

import jax
import jax.numpy as jnp
from jax import lax
from jax.experimental import pallas as pl
from jax.experimental.pallas import tpu as pltpu
from jax.experimental.pallas import tpu_sc as plsc
from jax.experimental.layout import Layout, with_layout_constraint

_VOCAB = 100000
_D = 16
_NSUB = 12
_E = _D * _NSUB
_BASE = 10000.0
_B, _T = 1024, 200
_BT = _B * _T

_PITCH_SIZE = 128
_POS_SIZE = 32
_BAR_ROWS = _T
_AUX_ROWS = _PITCH_SIZE + _POS_SIZE + _BAR_ROWS

_NC = 2
_NSC = 16
_NW = _NC * _NSC
_ROWS_PER_W = _B // _NW

_CA = 112
_CB = 88
_GROUPS = 13


def _aux_body(bias_ref, out_ref):
    rows = lax.broadcasted_iota(jnp.int32, (_AUX_ROWS, _E), 0)
    cols = lax.broadcasted_iota(jnp.int32, (_AUX_ROWS, _E), 1)
    d = cols % _D
    exponent = 2.0 * jnp.floor(d.astype(jnp.float32) / 2.0) / float(_D)
    rate = jnp.exp(exponent * (-jnp.log(jnp.float32(_BASE))))
    is_pos = (rows >= _PITCH_SIZE) & (rows < _PITCH_SIZE + _POS_SIZE)
    is_bar = rows >= _PITCH_SIZE + _POS_SIZE
    val = jnp.where(
        is_bar, rows - (_PITCH_SIZE + _POS_SIZE),
        jnp.where(is_pos, rows - _PITCH_SIZE, rows)).astype(jnp.float32)
    ang = val * rate
    enc = jnp.where(d % 2 == 0, jnp.sin(ang), jnp.cos(ang))
    pitch_b = bias_ref[0:1, :]
    pos_b = bias_ref[1:2, :]
    bar_b = bias_ref[2:3, :]
    bias = jnp.where(is_bar, bar_b, jnp.where(is_pos, pos_b, pitch_b))
    out_ref[:, :] = enc + bias


def _build_aux(biases):
    return pl.pallas_call(
        _aux_body,
        out_shape=jax.ShapeDtypeStruct((_AUX_ROWS, _E), jnp.float32),
    )(biases)


_TOK_W = _T * _ROWS_PER_W
_CHUNK = 128
_NCHUNK = _TOK_W // _CHUNK
_NBUF = 4


def _sc_body(idx_hbm, table_hbm, aux_hbm, out_hbm,
             idx_all, rb0, rb1, rb2, rb3, auxbuf,
             gs0, gs1, gs2, gs3, ws0, ws1, ws2, ws3, osem):
    bufs = (rb0, rb1, rb2, rb3)
    gsems = (gs0, gs1, gs2, gs3)
    wsems = (ws0, ws1, ws2, ws3)
    cid = lax.axis_index("c")
    sid = lax.axis_index("s")
    wid = sid * _NC + cid
    lanes = lax.iota(jnp.int32, 16)
    tok0 = pl.multiple_of(wid * _TOK_W, 8)

    pltpu.sync_copy(idx_hbm.at[pl.ds(tok0, _TOK_W)],
                    idx_all.at[pl.ds(0, _TOK_W)])
    idx_all[pl.ds(_TOK_W, 16)] = jnp.zeros((16,), jnp.int32)

    def g_start(c, s):
        pltpu.async_copy(
            table_hbm.at[idx_all.at[pl.ds(c * _CHUNK, _CHUNK)]],
            bufs[s], gsems[s])

    def g_wait(s):
        pltpu.make_async_copy(
            table_hbm.at[idx_all.at[pl.ds(0, _CHUNK)]],
            bufs[s], gsems[s]).wait()

    def w_start(c, s):
        pltpu.async_copy(
            bufs[s], out_hbm.at[pl.ds(tok0 + c * _CHUNK, _CHUNK)], wsems[s])

    def w_wait(s):
        pltpu.make_async_copy(
            bufs[s], out_hbm.at[pl.ds(tok0, _CHUNK)], wsems[s]).wait()

    g_start(0, 0)
    g_start(1, 1)

    def chunk4_body(c4, _):
        for s in range(_NBUF):
            d = c4 * _NBUF + s
            s2 = (s + 2) % _NBUF

            @pl.when(d >= 2)
            def _():
                w_wait(s2)

            @pl.when(d + 2 < _NCHUNK)
            def _():
                g_start(d + 2, s2)

            g_wait(s)
            w_start(d, s)
        return 0

    lax.fori_loop(0, _NCHUNK // _NBUF, chunk4_body, 0)
    for d in range(_NCHUNK - _NCHUNK % _NBUF, _NCHUNK):
        s = d % _NBUF
        s2 = (s + 2) % _NBUF
        w_wait(s2)
        g_wait(s)
        w_start(d, s)
    for d in range(_NCHUNK - 2, _NCHUNK):
        w_wait(d % _NBUF)

    def row_body(r, _):
        rb = pl.multiple_of(r * _T, 8)
        carry = jnp.int32(0)
        for g in range(_GROUPS):
            iv = idx_all[pl.ds(rb + g * 16, 16)]
            valid = lanes < (16 if g < 12 else 8)
            barm = (iv == 128) & valid
            bar_ones = jnp.where(barm, 1, 0)
            bar_idx = carry + jnp.cumsum(bar_ones) - 1
            carry = carry + jnp.sum(bar_ones)
            maskv = (iv < 161) & valid
            auxi = jnp.where(iv == 128, 160 + bar_idx,
                             jnp.where(iv < 128, iv, iv - 1))
            posg = tok0 + rb + g * 16 + lanes
            cnt = jnp.sum(jnp.where(maskv, 1, 0))

            @pl.when(cnt > 0)
            def _():
                lanestar = jnp.max(jnp.where(maskv, lanes, -1))
                onehot = jnp.where(lanes == lanestar, 1, 0)
                astar = jnp.sum(onehot * auxi)
                pstar = jnp.sum(onehot * posg)
                auxf = jnp.where(maskv, auxi, astar)
                posf = jnp.where(maskv, posg, pstar)
                pltpu.async_copy(aux_hbm.at[auxf], auxbuf, osem).wait()
                pltpu.async_copy(auxbuf, out_hbm.at[posf], osem).wait()
        return 0

    lax.fori_loop(0, _ROWS_PER_W, row_body, 0)


_sc_gather = functools.partial(
    pl.kernel,
    mesh=plsc.VectorSubcoreMesh(core_axis_name="c", subcore_axis_name="s"),
    out_type=jax.ShapeDtypeStruct((_BT, _E), jnp.float32),
    compiler_params=pltpu.CompilerParams(
        needs_layout_passes=False, use_tc_tiling_on_sc=False),
    scratch_types=[
        pltpu.VMEM((_TOK_W + 16,), jnp.int32),
        pltpu.VMEM((_CHUNK, _E), jnp.float32),
        pltpu.VMEM((_CHUNK, _E), jnp.float32),
        pltpu.VMEM((_CHUNK, _E), jnp.float32),
        pltpu.VMEM((_CHUNK, _E), jnp.float32),
        pltpu.VMEM((16, _E), jnp.float32),
        pltpu.SemaphoreType.DMA,
        pltpu.SemaphoreType.DMA,
        pltpu.SemaphoreType.DMA,
        pltpu.SemaphoreType.DMA,
        pltpu.SemaphoreType.DMA,
        pltpu.SemaphoreType.DMA,
        pltpu.SemaphoreType.DMA,
        pltpu.SemaphoreType.DMA,
        pltpu.SemaphoreType.DMA,
    ],
)(_sc_body)


def kernel(idx, table, pitch_bias, pos_bias, bar_bias):
    biases = jnp.concatenate([
        pitch_bias.reshape(1, _E),
        pos_bias.reshape(1, _E),
        bar_bias.reshape(1, _E),
    ], axis=0)
    aux = _build_aux(biases)
    table_lin = with_layout_constraint(
        table, Layout(major_to_minor=(0, 1), tiling=((8,), (1024,))))
    out = _sc_gather(idx.reshape(_BT).astype(jnp.int32), table_lin, aux)
    return out.reshape(_B, _T, _E)
